# baseline (speedup 1.0000x reference)
import jax
import jax.numpy as jnp
from jax import lax
from jax.experimental import pallas as pl
from jax.experimental.pallas import tpu as pltpu, tpu_sc as plsc

_S, _B, _T, _C = 2, 8, 500, 2048
_N_DEN, _A_DEN = 512, 8192
_N_NUM, _A_NUM = 128, 512
_DEN_SCALE = 1.0
_L = 16
_NC, _NS = 2, 16
_NW = _NC * _NS
_LN2 = 0.6931471805599453


def _sc_body(llh_ref, seql_ref, dsrc_ref, ddst_ref, dpdf_ref, dw_ref,
             dstart_ref, dfinal_ref, nsrc_ref, ndst_ref, npdf_ref, nw_ref,
             nstart_ref, nfinal_ref, out_ref,
             src_v, dst_v, pdf_v, pack_v, w_v, p_v, pnew_v, final_v, llh_v,
             seql_v, out_v, sem0, sem1):
    cid = lax.axis_index("c")
    sid = lax.axis_index("s")
    wid = sid * _NC + cid
    iota = lax.iota(jnp.int32, _L)
    zero16 = jnp.zeros((_L,), jnp.float32)

    pltpu.sync_copy(seql_ref, seql_v)

    def _pack(n_groups):
        def body(g, _):
            sl = pl.ds(g * _L, _L)
            pack_v[sl] = src_v[sl] | (dst_v[sl] << 9) | (pdf_v[sl] << 18)
            return 0
        lax.fori_loop(0, n_groups, body, 0)

    def _renorm_zero(n_groups, poff):
        def mx_body(g, mx):
            return jnp.maximum(mx, pnew_v[pl.ds(poff + g * _L, _L)])
        mx = lax.fori_loop(0, n_groups, mx_body, zero16)
        e_vec = (plsc.bitcast(mx, jnp.int32) >> 23) - 127
        e = jnp.max(e_vec.astype(jnp.float32)).astype(jnp.int32)
        scale = plsc.bitcast(jnp.broadcast_to(127 - e, (_L,)) << 23,
                             jnp.float32)

        def upd_body(g, _):
            sl = pl.ds(poff + g * _L, _L)
            p_v[sl] = pnew_v[sl] * scale
            pnew_v[sl] = zero16
            return 0
        lax.fori_loop(0, n_groups, upd_body, 0)
        return e

    def _zero_pnew(n_groups):
        def body(g, _):
            pnew_v[pl.ds(g * _L, _L)] = zero16
            return 0
        lax.fori_loop(0, n_groups, body, 0)

    def _final_lin(n_groups, poff):
        def body(g, acc):
            sl = pl.ds(poff + g * _L, _L)
            return acc + p_v[sl] * jnp.exp(final_v[sl])
        return jnp.sum(lax.fori_loop(0, n_groups, body, zero16))

    def _myseq(b):
        seqv = seql_v[...]
        return jnp.max(
            jnp.where(iota == b, seqv, 0).astype(jnp.float32)
        ).astype(jnp.int32)

    def _llh_wait_prefetch(t, base_row, myseq):
        par = jnp.bitwise_and(t, 1)

        @pl.when(par == 0)
        def _():
            pltpu.make_async_copy(llh_ref.at[base_row],
                                  llh_v.at[pl.ds(0, _C)], sem0).wait()

            @pl.when(t + 1 < myseq)
            def _():
                pltpu.make_async_copy(llh_ref.at[base_row + t + 1],
                                      llh_v.at[pl.ds(_C, _C)], sem1).start()

        @pl.when(par == 1)
        def _():
            pltpu.make_async_copy(llh_ref.at[base_row],
                                  llh_v.at[pl.ds(_C, _C)], sem1).wait()

            @pl.when(t + 1 < myseq)
            def _():
                pltpu.make_async_copy(llh_ref.at[base_row + t + 1],
                                      llh_v.at[pl.ds(0, _C)], sem0).start()

        return par * _C

    @pl.when(wid < _L)
    def _den():
        s = wid // _B
        b = wid % _B
        pltpu.sync_copy(dsrc_ref, src_v)
        pltpu.sync_copy(ddst_ref, dst_v)
        pltpu.sync_copy(dpdf_ref, pdf_v)
        pltpu.sync_copy(dw_ref, w_v)
        pltpu.sync_copy(dfinal_ref, final_v)
        pltpu.sync_copy(dstart_ref, pnew_v)
        _pack(_A_DEN // _L)

        def p_init(g, _):
            sl = pl.ds(g * _L, _L)
            p_v[sl] = jnp.exp(pnew_v[sl])
            return 0
        lax.fori_loop(0, _N_DEN // _L, p_init, 0)
        _zero_pnew(_N_DEN // _L)

        base_row = (s * _B + b) * _T
        myseq = _myseq(b)
        pltpu.make_async_copy(llh_ref.at[base_row],
                              llh_v.at[pl.ds(0, _C)], sem0).start()

        def t_body(t, e_tot):
            coff = _llh_wait_prefetch(t, base_row, myseq)

            def arc_body(g, _):
                for u in range(4):
                    sl = pl.ds((g * 4 + u) * _L, _L)
                    pk = pack_v[sl]
                    pv = plsc.load_gather(p_v, [pk & 511])
                    ov = plsc.load_gather(llh_v, [(pk >> 18) + coff])
                    val = pv * jnp.exp(w_v[sl] + ov)
                    plsc.addupdate_scatter(pnew_v, [(pk >> 9) & 511], val)
                return 0
            lax.fori_loop(0, _A_DEN // _L // 4, arc_body, 0)
            return e_tot + _renorm_zero(_N_DEN // _L, 0)

        e_tot = lax.fori_loop(0, myseq, t_body, jnp.int32(0))
        lin = _final_lin(_N_DEN // _L, 0)
        res = jnp.where(iota == 0, lin,
                        jnp.where(iota == 1, e_tot.astype(jnp.float32), 0.0))
        out_v[...] = res
        pltpu.sync_copy(out_v, out_ref.at[wid])

    @pl.when(wid >= _L)
    def _num():
        w2 = wid - _L
        s = w2 // _B
        b = w2 % _B
        for j in range(2):
            r = j * _B + b
            aoff = j * _A_NUM
            poff = j * _N_NUM
            pltpu.sync_copy(nsrc_ref.at[r], src_v.at[pl.ds(aoff, _A_NUM)])
            pltpu.sync_copy(ndst_ref.at[r], dst_v.at[pl.ds(aoff, _A_NUM)])
            pltpu.sync_copy(npdf_ref.at[r], pdf_v.at[pl.ds(aoff, _A_NUM)])
            pltpu.sync_copy(nw_ref.at[r], w_v.at[pl.ds(aoff, _A_NUM)])
            pltpu.sync_copy(nstart_ref.at[r], pnew_v.at[pl.ds(poff, _N_NUM)])
            pltpu.sync_copy(nfinal_ref.at[r], final_v.at[pl.ds(poff, _N_NUM)])
        _pack(2 * _A_NUM // _L)

        def p_init(g, _):
            sl = pl.ds(g * _L, _L)
            p_v[sl] = jnp.exp(pnew_v[sl])
            return 0
        lax.fori_loop(0, 2 * _N_NUM // _L, p_init, 0)
        _zero_pnew(2 * _N_NUM // _L)

        base_row = (s * _B + b) * _T
        myseq = _myseq(b)
        pltpu.make_async_copy(llh_ref.at[base_row],
                              llh_v.at[pl.ds(0, _C)], sem0).start()

        def t_body(t, es):
            coff = _llh_wait_prefetch(t, base_row, myseq)
            new_es = []
            for j in range(2):
                aoff = j * _A_NUM
                poff = j * _N_NUM

                def arc_body(g, _, aoff=aoff, poff=poff):
                    for u in range(4):
                        sl = pl.ds(aoff + (g * 4 + u) * _L, _L)
                        pk = pack_v[sl]
                        pv = plsc.load_gather(p_v, [(pk & 511) + poff])
                        ov = plsc.load_gather(llh_v, [(pk >> 18) + coff])
                        val = pv * jnp.exp(w_v[sl] + ov)
                        plsc.addupdate_scatter(
                            pnew_v, [((pk >> 9) & 511) + poff], val)
                    return 0
                lax.fori_loop(0, _A_NUM // _L // 4, arc_body, 0)
                new_es.append(es[j] + _renorm_zero(_N_NUM // _L, poff))
            return tuple(new_es)

        e0, e1 = lax.fori_loop(0, myseq, t_body,
                               (jnp.int32(0), jnp.int32(0)))
        lin0 = _final_lin(_N_NUM // _L, 0)
        lin1 = _final_lin(_N_NUM // _L, _N_NUM)
        res = jnp.where(iota == 0, lin0,
              jnp.where(iota == 1, e0.astype(jnp.float32),
              jnp.where(iota == 2, lin1,
              jnp.where(iota == 3, e1.astype(jnp.float32), 0.0))))
        out_v[...] = res
        pltpu.sync_copy(out_v, out_ref.at[wid])


def kernel(est_llhs, seqlengths, den_src, den_dst, den_pdf, den_weight,
           den_start, den_final, num_src, num_dst, num_pdf, num_weight,
           num_start, num_final):
    llh_rows = est_llhs.reshape(_S * _B * _T, _C)
    seql16 = jnp.zeros((16,), jnp.int32).at[:_B].set(
        seqlengths.astype(jnp.int32))
    mesh = plsc.VectorSubcoreMesh(core_axis_name="c", subcore_axis_name="s",
                                  num_cores=_NC, num_subcores=_NS)
    scratch = [
        pltpu.VMEM((_A_DEN,), jnp.int32),
        pltpu.VMEM((_A_DEN,), jnp.int32),
        pltpu.VMEM((_A_DEN,), jnp.int32),
        pltpu.VMEM((_A_DEN,), jnp.int32),
        pltpu.VMEM((_A_DEN,), jnp.float32),
        pltpu.VMEM((_N_DEN,), jnp.float32),
        pltpu.VMEM((_N_DEN,), jnp.float32),
        pltpu.VMEM((_N_DEN,), jnp.float32),
        pltpu.VMEM((2 * _C,), jnp.float32),
        pltpu.VMEM((16,), jnp.int32),
        pltpu.VMEM((16,), jnp.float32),
        pltpu.SemaphoreType.DMA,
        pltpu.SemaphoreType.DMA,
    ]
    run = pl.kernel(_sc_body,
                    out_type=jax.ShapeDtypeStruct((_NW, 16), jnp.float32),
                    mesh=mesh, scratch_types=scratch,
                    compiler_params=pltpu.CompilerParams(
                        needs_layout_passes=False))
    out = run(llh_rows, seql16,
              den_src.astype(jnp.int32), den_dst.astype(jnp.int32),
              den_pdf.astype(jnp.int32), den_weight,
              den_start, den_final,
              num_src.reshape(_S * _B, _A_NUM).astype(jnp.int32),
              num_dst.reshape(_S * _B, _A_NUM).astype(jnp.int32),
              num_pdf.reshape(_S * _B, _A_NUM).astype(jnp.int32),
              num_weight.reshape(_S * _B, _A_NUM),
              num_start.reshape(_S * _B, _N_NUM),
              num_final.reshape(_S * _B, _N_NUM))

    den = out[:_L]
    den_llh = (jnp.log(den[:, 0]) + den[:, 1] * _LN2).reshape(_S, _B).T
    num = out[_L:]
    num_ll = (jnp.log(num[:, jnp.array([0, 2])])
              + num[:, jnp.array([1, 3])] * _LN2).reshape(_S, _B, 2)
    nlp0 = jnp.stack([num_ll[0, :, 0], num_ll[1, :, 1]], axis=1)
    nlp1 = jnp.stack([num_ll[0, :, 1], num_ll[1, :, 0]], axis=1)
    use1 = jnp.sum(nlp1, axis=1) > jnp.sum(nlp0, axis=1)
    num_llh = jnp.where(use1[:, None], nlp1, nlp0)
    loss = -(num_llh - _DEN_SCALE * den_llh)
    return loss.sum()


# speedup vs baseline: 1.3052x; 1.3052x over previous
import jax
import jax.numpy as jnp
from jax import lax
from jax.experimental import pallas as pl
from jax.experimental.pallas import tpu as pltpu, tpu_sc as plsc

_S, _B, _T, _C = 2, 8, 500, 2048
_N_DEN, _A_DEN = 512, 8192
_N_NUM, _A_NUM = 128, 512
_DEN_SCALE = 1.0
_L = 16
_NC, _NS = 2, 16
_NW = _NC * _NS
_LN2 = 0.6931471805599453


def _sc_body(llh_ref, seql_ref, dsrc_ref, ddst_ref, dpdf_ref, dw_ref,
             dstart_ref, dfinal_ref, nsrc_ref, ndst_ref, npdf_ref, nw_ref,
             nstart_ref, nfinal_ref, out_ref,
             src_v, dst_v, pdf_v, pack_v, w_v, p_v, pnew_v, final_v, llh_v,
             eobs_v, seql_v, out_v, sem0, sem1):
    cid = lax.axis_index("c")
    sid = lax.axis_index("s")
    wid = sid * _NC + cid
    iota = lax.iota(jnp.int32, _L)
    zero16 = jnp.zeros((_L,), jnp.float32)

    pltpu.sync_copy(seql_ref, seql_v)

    def _pack(n_groups):
        def body(g, _):
            sl = pl.ds(g * _L, _L)
            pack_v[sl] = src_v[sl] | (dst_v[sl] << 9) | (pdf_v[sl] << 18)
            return 0
        lax.fori_loop(0, n_groups, body, 0)

    def _renorm_zero(n_groups, poff):
        def mx_body(g, mx):
            return jnp.maximum(mx, pnew_v[pl.ds(poff + g * _L, _L)])
        mx = lax.fori_loop(0, n_groups, mx_body, zero16)
        e_vec = (plsc.bitcast(mx, jnp.int32) >> 23) - 127
        e = jnp.max(e_vec.astype(jnp.float32)).astype(jnp.int32)
        scale = plsc.bitcast(jnp.broadcast_to(127 - e, (_L,)) << 23,
                             jnp.float32)

        def upd_body(g, _):
            sl = pl.ds(poff + g * _L, _L)
            p_v[sl] = pnew_v[sl] * scale
            pnew_v[sl] = zero16
            return 0
        lax.fori_loop(0, n_groups, upd_body, 0)
        return e

    def _zero_pnew(n_groups):
        def body(g, _):
            pnew_v[pl.ds(g * _L, _L)] = zero16
            return 0
        lax.fori_loop(0, n_groups, body, 0)

    def _final_lin(n_groups, poff):
        def body(g, acc):
            sl = pl.ds(poff + g * _L, _L)
            return acc + p_v[sl] * jnp.exp(final_v[sl])
        return jnp.sum(lax.fori_loop(0, n_groups, body, zero16))

    def _myseq(b):
        seqv = seql_v[...]
        return jnp.max(
            jnp.where(iota == b, seqv, 0).astype(jnp.float32)
        ).astype(jnp.int32)

    def _llh_wait_prefetch(t, base_row, myseq):
        par = jnp.bitwise_and(t, 1)

        @pl.when(par == 0)
        def _():
            pltpu.make_async_copy(llh_ref.at[base_row],
                                  llh_v.at[pl.ds(0, _C)], sem0).wait()

            @pl.when(t + 1 < myseq)
            def _():
                pltpu.make_async_copy(llh_ref.at[base_row + t + 1],
                                      llh_v.at[pl.ds(_C, _C)], sem1).start()

        @pl.when(par == 1)
        def _():
            pltpu.make_async_copy(llh_ref.at[base_row],
                                  llh_v.at[pl.ds(_C, _C)], sem1).wait()

            @pl.when(t + 1 < myseq)
            def _():
                pltpu.make_async_copy(llh_ref.at[base_row + t + 1],
                                      llh_v.at[pl.ds(0, _C)], sem0).start()

        return par * _C

    @pl.when(wid < _L)
    def _den():
        s = wid // _B
        b = wid % _B
        pltpu.sync_copy(dsrc_ref, src_v)
        pltpu.sync_copy(ddst_ref, dst_v)
        pltpu.sync_copy(dpdf_ref, pdf_v)
        pltpu.sync_copy(dw_ref, w_v)
        pltpu.sync_copy(dfinal_ref, final_v)
        pltpu.sync_copy(dstart_ref, pnew_v)
        _pack(_A_DEN // _L)

        def w_exp(g, _):
            sl = pl.ds(g * _L, _L)
            w_v[sl] = jnp.exp(w_v[sl])
            return 0
        lax.fori_loop(0, _A_DEN // _L, w_exp, 0)

        def p_init(g, _):
            sl = pl.ds(g * _L, _L)
            p_v[sl] = jnp.exp(pnew_v[sl])
            return 0
        lax.fori_loop(0, _N_DEN // _L, p_init, 0)
        _zero_pnew(_N_DEN // _L)

        base_row = (s * _B + b) * _T
        myseq = _myseq(b)
        pltpu.make_async_copy(llh_ref.at[base_row],
                              llh_v.at[pl.ds(0, _C)], sem0).start()

        def t_body(t, e_tot):
            coff = _llh_wait_prefetch(t, base_row, myseq)

            def eobs_body(g, _):
                sl = pl.ds(g * _L, _L)
                eobs_v[sl] = jnp.exp(llh_v[pl.ds(coff + g * _L, _L)])
                return 0
            lax.fori_loop(0, _C // _L, eobs_body, 0)

            def arc_body(g, _):
                for u in range(4):
                    sl = pl.ds((g * 4 + u) * _L, _L)
                    pk = pack_v[sl]
                    pv = plsc.load_gather(p_v, [pk & 511])
                    ov = plsc.load_gather(eobs_v, [pk >> 18])
                    val = pv * w_v[sl] * ov
                    plsc.addupdate_scatter(pnew_v, [(pk >> 9) & 511], val)
                return 0
            lax.fori_loop(0, _A_DEN // _L // 4, arc_body, 0)
            return e_tot + _renorm_zero(_N_DEN // _L, 0)

        e_tot = lax.fori_loop(0, myseq, t_body, jnp.int32(0))
        lin = _final_lin(_N_DEN // _L, 0)
        res = jnp.where(iota == 0, lin,
                        jnp.where(iota == 1, e_tot.astype(jnp.float32), 0.0))
        out_v[...] = res
        pltpu.sync_copy(out_v, out_ref.at[wid])

    @pl.when(wid >= _L)
    def _num():
        w2 = wid - _L
        s = w2 // _B
        b = w2 % _B
        for j in range(2):
            r = j * _B + b
            aoff = j * _A_NUM
            poff = j * _N_NUM
            pltpu.sync_copy(nsrc_ref.at[r], src_v.at[pl.ds(aoff, _A_NUM)])
            pltpu.sync_copy(ndst_ref.at[r], dst_v.at[pl.ds(aoff, _A_NUM)])
            pltpu.sync_copy(npdf_ref.at[r], pdf_v.at[pl.ds(aoff, _A_NUM)])
            pltpu.sync_copy(nw_ref.at[r], w_v.at[pl.ds(aoff, _A_NUM)])
            pltpu.sync_copy(nstart_ref.at[r], pnew_v.at[pl.ds(poff, _N_NUM)])
            pltpu.sync_copy(nfinal_ref.at[r], final_v.at[pl.ds(poff, _N_NUM)])
        _pack(2 * _A_NUM // _L)

        def p_init(g, _):
            sl = pl.ds(g * _L, _L)
            p_v[sl] = jnp.exp(pnew_v[sl])
            return 0
        lax.fori_loop(0, 2 * _N_NUM // _L, p_init, 0)
        _zero_pnew(2 * _N_NUM // _L)

        base_row = (s * _B + b) * _T
        myseq = _myseq(b)
        pltpu.make_async_copy(llh_ref.at[base_row],
                              llh_v.at[pl.ds(0, _C)], sem0).start()

        def t_body(t, es):
            coff = _llh_wait_prefetch(t, base_row, myseq)
            new_es = []
            for j in range(2):
                aoff = j * _A_NUM
                poff = j * _N_NUM

                def arc_body(g, _, aoff=aoff, poff=poff):
                    for u in range(4):
                        sl = pl.ds(aoff + (g * 4 + u) * _L, _L)
                        pk = pack_v[sl]
                        pv = plsc.load_gather(p_v, [(pk & 511) + poff])
                        ov = plsc.load_gather(llh_v, [(pk >> 18) + coff])
                        val = pv * jnp.exp(w_v[sl] + ov)
                        plsc.addupdate_scatter(
                            pnew_v, [((pk >> 9) & 511) + poff], val)
                    return 0
                lax.fori_loop(0, _A_NUM // _L // 4, arc_body, 0)
                new_es.append(es[j] + _renorm_zero(_N_NUM // _L, poff))
            return tuple(new_es)

        e0, e1 = lax.fori_loop(0, myseq, t_body,
                               (jnp.int32(0), jnp.int32(0)))
        lin0 = _final_lin(_N_NUM // _L, 0)
        lin1 = _final_lin(_N_NUM // _L, _N_NUM)
        res = jnp.where(iota == 0, lin0,
              jnp.where(iota == 1, e0.astype(jnp.float32),
              jnp.where(iota == 2, lin1,
              jnp.where(iota == 3, e1.astype(jnp.float32), 0.0))))
        out_v[...] = res
        pltpu.sync_copy(out_v, out_ref.at[wid])


def kernel(est_llhs, seqlengths, den_src, den_dst, den_pdf, den_weight,
           den_start, den_final, num_src, num_dst, num_pdf, num_weight,
           num_start, num_final):
    llh_rows = est_llhs.reshape(_S * _B * _T, _C)
    seql16 = jnp.zeros((16,), jnp.int32).at[:_B].set(
        seqlengths.astype(jnp.int32))
    mesh = plsc.VectorSubcoreMesh(core_axis_name="c", subcore_axis_name="s",
                                  num_cores=_NC, num_subcores=_NS)
    scratch = [
        pltpu.VMEM((_A_DEN,), jnp.int32),
        pltpu.VMEM((_A_DEN,), jnp.int32),
        pltpu.VMEM((_A_DEN,), jnp.int32),
        pltpu.VMEM((_A_DEN,), jnp.int32),
        pltpu.VMEM((_A_DEN,), jnp.float32),
        pltpu.VMEM((_N_DEN,), jnp.float32),
        pltpu.VMEM((_N_DEN,), jnp.float32),
        pltpu.VMEM((_N_DEN,), jnp.float32),
        pltpu.VMEM((2 * _C,), jnp.float32),
        pltpu.VMEM((_C,), jnp.float32),
        pltpu.VMEM((16,), jnp.int32),
        pltpu.VMEM((16,), jnp.float32),
        pltpu.SemaphoreType.DMA,
        pltpu.SemaphoreType.DMA,
    ]
    run = pl.kernel(_sc_body,
                    out_type=jax.ShapeDtypeStruct((_NW, 16), jnp.float32),
                    mesh=mesh, scratch_types=scratch,
                    compiler_params=pltpu.CompilerParams(
                        needs_layout_passes=False))
    out = run(llh_rows, seql16,
              den_src.astype(jnp.int32), den_dst.astype(jnp.int32),
              den_pdf.astype(jnp.int32), den_weight,
              den_start, den_final,
              num_src.reshape(_S * _B, _A_NUM).astype(jnp.int32),
              num_dst.reshape(_S * _B, _A_NUM).astype(jnp.int32),
              num_pdf.reshape(_S * _B, _A_NUM).astype(jnp.int32),
              num_weight.reshape(_S * _B, _A_NUM),
              num_start.reshape(_S * _B, _N_NUM),
              num_final.reshape(_S * _B, _N_NUM))

    den = out[:_L]
    den_llh = (jnp.log(den[:, 0]) + den[:, 1] * _LN2).reshape(_S, _B).T
    num = out[_L:]
    num_ll = (jnp.log(num[:, jnp.array([0, 2])])
              + num[:, jnp.array([1, 3])] * _LN2).reshape(_S, _B, 2)
    nlp0 = jnp.stack([num_ll[0, :, 0], num_ll[1, :, 1]], axis=1)
    nlp1 = jnp.stack([num_ll[0, :, 1], num_ll[1, :, 0]], axis=1)
    use1 = jnp.sum(nlp1, axis=1) > jnp.sum(nlp0, axis=1)
    num_llh = jnp.where(use1[:, None], nlp1, nlp0)
    loss = -(num_llh - _DEN_SCALE * den_llh)
    return loss.sum()


# speedup vs baseline: 1.3423x; 1.0284x over previous
import jax
import jax.numpy as jnp
from jax import lax
from jax.experimental import pallas as pl
from jax.experimental.pallas import tpu as pltpu, tpu_sc as plsc

_S, _B, _T, _C = 2, 8, 500, 2048
_N_DEN, _A_DEN = 512, 8192
_N_NUM, _A_NUM = 128, 512
_DEN_SCALE = 1.0
_L = 16
_NC, _NS = 2, 16
_NW = _NC * _NS
_LN2 = 0.6931471805599453


def _sc_body(llh_ref, seql_ref, dsrc_ref, ddst_ref, dpdf_ref, dw_ref,
             dstart_ref, dfinal_ref, nsrc_ref, ndst_ref, npdf_ref, nw_ref,
             nstart_ref, nfinal_ref, out_ref,
             src_v, dst_v, pdf_v, pack_v, w_v, p_v, pnew_v, final_v, llh_v,
             eobs_v, seql_v, out_v, sem0, sem1):
    cid = lax.axis_index("c")
    sid = lax.axis_index("s")
    wid = sid * _NC + cid
    iota = lax.iota(jnp.int32, _L)
    zero16 = jnp.zeros((_L,), jnp.float32)

    pltpu.sync_copy(seql_ref, seql_v)

    def _pack(n_groups):
        def body(g, _):
            sl = pl.ds(g * _L, _L)
            pack_v[sl] = src_v[sl] | (dst_v[sl] << 9) | (pdf_v[sl] << 18)
            return 0
        lax.fori_loop(0, n_groups, body, 0)

    def _renorm_zero(n_groups, poff):
        def mx_body(g, mx):
            return jnp.maximum(mx, pnew_v[pl.ds(poff + g * _L, _L)])
        mx = lax.fori_loop(0, n_groups, mx_body, zero16)
        e_vec = (plsc.bitcast(mx, jnp.int32) >> 23) - 127
        e = jnp.max(e_vec.astype(jnp.float32)).astype(jnp.int32)
        scale = plsc.bitcast(jnp.broadcast_to(127 - e, (_L,)) << 23,
                             jnp.float32)

        def upd_body(g, _):
            sl = pl.ds(poff + g * _L, _L)
            p_v[sl] = pnew_v[sl] * scale
            pnew_v[sl] = zero16
            return 0
        lax.fori_loop(0, n_groups, upd_body, 0)
        return e

    def _zero_pnew(n_groups):
        def body(g, _):
            pnew_v[pl.ds(g * _L, _L)] = zero16
            return 0
        lax.fori_loop(0, n_groups, body, 0)

    def _final_lin(n_groups, poff):
        def body(g, acc):
            sl = pl.ds(poff + g * _L, _L)
            return acc + p_v[sl] * jnp.exp(final_v[sl])
        return jnp.sum(lax.fori_loop(0, n_groups, body, zero16))

    def _myseq(b):
        seqv = seql_v[...]
        return jnp.max(
            jnp.where(iota == b, seqv, 0).astype(jnp.float32)
        ).astype(jnp.int32)

    def _llh_wait_prefetch(t, base_row, myseq):
        pltpu.sync_copy(llh_ref.at[base_row + t], llh_v.at[pl.ds(0, _C)])
        return 0

    @pl.when(wid < _L)
    def _den():
        s = wid // _B
        b = wid % _B
        pltpu.sync_copy(dsrc_ref, src_v)
        pltpu.sync_copy(ddst_ref, dst_v)
        pltpu.sync_copy(dpdf_ref, pdf_v)
        pltpu.sync_copy(dw_ref, w_v)
        pltpu.sync_copy(dfinal_ref, final_v)
        pltpu.sync_copy(dstart_ref, pnew_v)
        _pack(_A_DEN // _L)

        def w_exp(g, _):
            sl = pl.ds(g * _L, _L)
            w_v[sl] = jnp.exp(w_v[sl])
            return 0
        lax.fori_loop(0, _A_DEN // _L, w_exp, 0)

        def p_init(g, _):
            sl = pl.ds(g * _L, _L)
            p_v[sl] = jnp.exp(pnew_v[sl])
            return 0
        lax.fori_loop(0, _N_DEN // _L, p_init, 0)
        _zero_pnew(_N_DEN // _L)

        base_row = (s * _B + b) * _T
        myseq = _myseq(b)

        def t_body(t, e_tot):
            coff = _llh_wait_prefetch(t, base_row, myseq)

            def eobs_body(g, _):
                sl = pl.ds(g * _L, _L)
                eobs_v[sl] = jnp.exp(llh_v[pl.ds(coff + g * _L, _L)])
                return 0
            lax.fori_loop(0, _C // _L, eobs_body, 0)

            def arc_body(g, _):
                for u in range(4):
                    sl = pl.ds((g * 4 + u) * _L, _L)
                    pk = pack_v[sl]
                    pv = plsc.load_gather(p_v, [pk & 511])
                    ov = plsc.load_gather(eobs_v, [pk >> 18])
                    val = pv * w_v[sl] * ov
                    plsc.addupdate_scatter(pnew_v, [(pk >> 9) & 511], val)
                return 0
            lax.fori_loop(0, _A_DEN // _L // 4, arc_body, 0)
            return e_tot + _renorm_zero(_N_DEN // _L, 0)

        e_tot = lax.fori_loop(0, myseq, t_body, jnp.int32(0))
        lin = _final_lin(_N_DEN // _L, 0)
        res = jnp.where(iota == 0, lin,
                        jnp.where(iota == 1, e_tot.astype(jnp.float32), 0.0))
        out_v[...] = res
        pltpu.sync_copy(out_v, out_ref.at[wid])

    @pl.when(wid >= _L)
    def _num():
        w2 = wid - _L
        s = w2 // _B
        b = w2 % _B
        for j in range(2):
            r = j * _B + b
            aoff = j * _A_NUM
            poff = j * _N_NUM
            pltpu.sync_copy(nsrc_ref.at[r], src_v.at[pl.ds(aoff, _A_NUM)])
            pltpu.sync_copy(ndst_ref.at[r], dst_v.at[pl.ds(aoff, _A_NUM)])
            pltpu.sync_copy(npdf_ref.at[r], pdf_v.at[pl.ds(aoff, _A_NUM)])
            pltpu.sync_copy(nw_ref.at[r], w_v.at[pl.ds(aoff, _A_NUM)])
            pltpu.sync_copy(nstart_ref.at[r], pnew_v.at[pl.ds(poff, _N_NUM)])
            pltpu.sync_copy(nfinal_ref.at[r], final_v.at[pl.ds(poff, _N_NUM)])
        _pack(2 * _A_NUM // _L)

        def p_init(g, _):
            sl = pl.ds(g * _L, _L)
            p_v[sl] = jnp.exp(pnew_v[sl])
            return 0
        lax.fori_loop(0, 2 * _N_NUM // _L, p_init, 0)
        _zero_pnew(2 * _N_NUM // _L)

        base_row = (s * _B + b) * _T
        myseq = _myseq(b)

        def t_body(t, es):
            coff = _llh_wait_prefetch(t, base_row, myseq)
            new_es = []
            for j in range(2):
                aoff = j * _A_NUM
                poff = j * _N_NUM

                def arc_body(g, _, aoff=aoff, poff=poff):
                    for u in range(4):
                        sl = pl.ds(aoff + (g * 4 + u) * _L, _L)
                        pk = pack_v[sl]
                        pv = plsc.load_gather(p_v, [(pk & 511) + poff])
                        ov = plsc.load_gather(llh_v, [(pk >> 18) + coff])
                        val = pv * jnp.exp(w_v[sl] + ov)
                        plsc.addupdate_scatter(
                            pnew_v, [((pk >> 9) & 511) + poff], val)
                    return 0
                lax.fori_loop(0, _A_NUM // _L // 4, arc_body, 0)
                new_es.append(es[j] + _renorm_zero(_N_NUM // _L, poff))
            return tuple(new_es)

        e0, e1 = lax.fori_loop(0, myseq, t_body,
                               (jnp.int32(0), jnp.int32(0)))
        lin0 = _final_lin(_N_NUM // _L, 0)
        lin1 = _final_lin(_N_NUM // _L, _N_NUM)
        res = jnp.where(iota == 0, lin0,
              jnp.where(iota == 1, e0.astype(jnp.float32),
              jnp.where(iota == 2, lin1,
              jnp.where(iota == 3, e1.astype(jnp.float32), 0.0))))
        out_v[...] = res
        pltpu.sync_copy(out_v, out_ref.at[wid])


def kernel(est_llhs, seqlengths, den_src, den_dst, den_pdf, den_weight,
           den_start, den_final, num_src, num_dst, num_pdf, num_weight,
           num_start, num_final):
    llh_rows = est_llhs.reshape(_S * _B * _T, _C)
    seql16 = jnp.zeros((16,), jnp.int32).at[:_B].set(
        seqlengths.astype(jnp.int32))
    mesh = plsc.VectorSubcoreMesh(core_axis_name="c", subcore_axis_name="s",
                                  num_cores=_NC, num_subcores=_NS)
    scratch = [
        pltpu.VMEM((_A_DEN,), jnp.int32),
        pltpu.VMEM((_A_DEN,), jnp.int32),
        pltpu.VMEM((_A_DEN,), jnp.int32),
        pltpu.VMEM((_A_DEN,), jnp.int32),
        pltpu.VMEM((_A_DEN,), jnp.float32),
        pltpu.VMEM((_N_DEN,), jnp.float32),
        pltpu.VMEM((_N_DEN,), jnp.float32),
        pltpu.VMEM((_N_DEN,), jnp.float32),
        pltpu.VMEM((2 * _C,), jnp.float32),
        pltpu.VMEM((_C,), jnp.float32),
        pltpu.VMEM((16,), jnp.int32),
        pltpu.VMEM((16,), jnp.float32),
        pltpu.SemaphoreType.DMA,
        pltpu.SemaphoreType.DMA,
    ]
    run = pl.kernel(_sc_body,
                    out_type=jax.ShapeDtypeStruct((_NW, 16), jnp.float32),
                    mesh=mesh, scratch_types=scratch,
                    compiler_params=pltpu.CompilerParams(
                        needs_layout_passes=False))
    out = run(llh_rows, seql16,
              den_src.astype(jnp.int32), den_dst.astype(jnp.int32),
              den_pdf.astype(jnp.int32), den_weight,
              den_start, den_final,
              num_src.reshape(_S * _B, _A_NUM).astype(jnp.int32),
              num_dst.reshape(_S * _B, _A_NUM).astype(jnp.int32),
              num_pdf.reshape(_S * _B, _A_NUM).astype(jnp.int32),
              num_weight.reshape(_S * _B, _A_NUM),
              num_start.reshape(_S * _B, _N_NUM),
              num_final.reshape(_S * _B, _N_NUM))

    den = out[:_L]
    den_llh = (jnp.log(den[:, 0]) + den[:, 1] * _LN2).reshape(_S, _B).T
    num = out[_L:]
    num_ll = (jnp.log(num[:, jnp.array([0, 2])])
              + num[:, jnp.array([1, 3])] * _LN2).reshape(_S, _B, 2)
    nlp0 = jnp.stack([num_ll[0, :, 0], num_ll[1, :, 1]], axis=1)
    nlp1 = jnp.stack([num_ll[0, :, 1], num_ll[1, :, 0]], axis=1)
    use1 = jnp.sum(nlp1, axis=1) > jnp.sum(nlp0, axis=1)
    num_llh = jnp.where(use1[:, None], nlp1, nlp0)
    loss = -(num_llh - _DEN_SCALE * den_llh)
    return loss.sum()


# speedup vs baseline: 3.4182x; 2.5465x over previous
import jax
import jax.numpy as jnp
from jax import lax
from jax.experimental import pallas as pl
from jax.experimental.pallas import tpu as pltpu, tpu_sc as plsc

_S, _B, _T, _C = 2, 8, 500, 2048
_N_DEN, _A_DEN = 512, 8192
_N_NUM, _A_NUM = 128, 512
_DEN_SCALE = 1.0
_L = 16
_NC, _NS = 2, 16
_NW = _NC * _NS
_LN2 = 0.6931471805599453


def _sc_body(llh_ref, seql_ref, dsrc_ref, ddst_ref, dpdf_ref, dw_ref,
             dstart_ref, dfinal_ref, nsrc_ref, ndst_ref, npdf_ref, nw_ref,
             nstart_ref, nfinal_ref, out_ref,
             src_v, dst_v, pdf_v, pack_v, w_v, p_v, pnew_v, final_v, llh_v,
             eobs_v, seql_v, out_v, sem0, sem1):
    cid = lax.axis_index("c")
    sid = lax.axis_index("s")
    wid = sid * _NC + cid
    iota = lax.iota(jnp.int32, _L)
    zero16 = jnp.zeros((_L,), jnp.float32)

    pltpu.sync_copy(seql_ref, seql_v)

    def _pack(n_groups):
        def body(g, _):
            sl = pl.ds(g * _L, _L)
            pack_v[sl] = src_v[sl] | (dst_v[sl] << 9) | (pdf_v[sl] << 18)
            return 0
        lax.fori_loop(0, n_groups, body, 0)

    def _renorm_zero(n_groups, poff):
        def mx_body(g, mx):
            return jnp.maximum(mx, pnew_v[pl.ds(poff + g * _L, _L)])
        mx = lax.fori_loop(0, n_groups, mx_body, zero16)
        e_vec = (plsc.bitcast(mx, jnp.int32) >> 23) - 127
        e = jnp.max(e_vec.astype(jnp.float32)).astype(jnp.int32)
        scale = plsc.bitcast(jnp.broadcast_to(127 - e, (_L,)) << 23,
                             jnp.float32)

        def upd_body(g, _):
            sl = pl.ds(poff + g * _L, _L)
            p_v[sl] = pnew_v[sl] * scale
            pnew_v[sl] = zero16
            return 0
        lax.fori_loop(0, n_groups, upd_body, 0)
        return e

    def _zero_pnew(n_groups):
        def body(g, _):
            pnew_v[pl.ds(g * _L, _L)] = zero16
            return 0
        lax.fori_loop(0, n_groups, body, 0)

    def _final_lin(n_groups, poff):
        def body(g, acc):
            sl = pl.ds(poff + g * _L, _L)
            return acc + p_v[sl] * jnp.exp(final_v[sl])
        return jnp.sum(lax.fori_loop(0, n_groups, body, zero16))

    def _myseq(b):
        seqv = seql_v[...]
        return jnp.max(
            jnp.where(iota == b, seqv, 0).astype(jnp.float32)
        ).astype(jnp.int32)

    def _llh_wait_prefetch(t, base_row, myseq):
        pltpu.sync_copy(llh_ref.at[base_row + t], llh_v.at[pl.ds(0, _C)])
        return 0

    @pl.when(wid < _L)
    def _den():
        s = wid // _B
        b = wid % _B
        pltpu.sync_copy(dsrc_ref, src_v)
        pltpu.sync_copy(ddst_ref, dst_v)
        pltpu.sync_copy(dpdf_ref, pdf_v)
        pltpu.sync_copy(dw_ref, w_v)
        pltpu.sync_copy(dfinal_ref, final_v)
        pltpu.sync_copy(dstart_ref, pnew_v)
        _pack(_A_DEN // _L)

        def w_exp(g, _):
            sl = pl.ds(g * _L, _L)
            w_v[sl] = jnp.exp(w_v[sl])
            return 0
        lax.fori_loop(0, _A_DEN // _L, w_exp, 0)

        def p_init(g, _):
            sl = pl.ds(g * _L, _L)
            p_v[sl] = jnp.exp(pnew_v[sl])
            return 0
        lax.fori_loop(0, _N_DEN // _L, p_init, 0)
        _zero_pnew(_N_DEN // _L)

        base_row = (s * _B + b) * _T
        myseq = _myseq(b)

        def t_body(t, e_tot):
            coff = _llh_wait_prefetch(t, base_row, myseq)

            @plsc.parallel_loop(0, _C // _L, unroll=4)
            def eobs_body(g):
                sl = pl.ds(g * _L, _L)
                eobs_v[sl] = jnp.exp(llh_v[pl.ds(coff + g * _L, _L)])

            @plsc.parallel_loop(0, _A_DEN // _L, unroll=4)
            def arc_body(g):
                sl = pl.ds(g * _L, _L)
                pk = pack_v[sl]
                pv = plsc.load_gather(p_v, [pk & 511])
                ov = plsc.load_gather(eobs_v, [pk >> 18])
                val = pv * w_v[sl] * ov
                plsc.addupdate_scatter(pnew_v, [(pk >> 9) & 511], val)

            return e_tot + _renorm_zero(_N_DEN // _L, 0)

        e_tot = lax.fori_loop(0, myseq, t_body, jnp.int32(0))
        lin = _final_lin(_N_DEN // _L, 0)
        res = jnp.where(iota == 0, lin,
                        jnp.where(iota == 1, e_tot.astype(jnp.float32), 0.0))
        out_v[...] = res
        pltpu.sync_copy(out_v, out_ref.at[wid])

    @pl.when(wid >= _L)
    def _num():
        w2 = wid - _L
        s = w2 // _B
        b = w2 % _B
        for j in range(2):
            r = j * _B + b
            aoff = j * _A_NUM
            poff = j * _N_NUM
            pltpu.sync_copy(nsrc_ref.at[r], src_v.at[pl.ds(aoff, _A_NUM)])
            pltpu.sync_copy(ndst_ref.at[r], dst_v.at[pl.ds(aoff, _A_NUM)])
            pltpu.sync_copy(npdf_ref.at[r], pdf_v.at[pl.ds(aoff, _A_NUM)])
            pltpu.sync_copy(nw_ref.at[r], w_v.at[pl.ds(aoff, _A_NUM)])
            pltpu.sync_copy(nstart_ref.at[r], pnew_v.at[pl.ds(poff, _N_NUM)])
            pltpu.sync_copy(nfinal_ref.at[r], final_v.at[pl.ds(poff, _N_NUM)])
        _pack(2 * _A_NUM // _L)

        def p_init(g, _):
            sl = pl.ds(g * _L, _L)
            p_v[sl] = jnp.exp(pnew_v[sl])
            return 0
        lax.fori_loop(0, 2 * _N_NUM // _L, p_init, 0)
        _zero_pnew(2 * _N_NUM // _L)

        base_row = (s * _B + b) * _T
        myseq = _myseq(b)

        def t_body(t, es):
            coff = _llh_wait_prefetch(t, base_row, myseq)
            new_es = []
            for j in range(2):
                aoff = j * _A_NUM
                poff = j * _N_NUM

                def arc_body(g, _, aoff=aoff, poff=poff):
                    for u in range(4):
                        sl = pl.ds(aoff + (g * 4 + u) * _L, _L)
                        pk = pack_v[sl]
                        pv = plsc.load_gather(p_v, [(pk & 511) + poff])
                        ov = plsc.load_gather(llh_v, [(pk >> 18) + coff])
                        val = pv * jnp.exp(w_v[sl] + ov)
                        plsc.addupdate_scatter(
                            pnew_v, [((pk >> 9) & 511) + poff], val)
                    return 0
                lax.fori_loop(0, _A_NUM // _L // 4, arc_body, 0)
                new_es.append(es[j] + _renorm_zero(_N_NUM // _L, poff))
            return tuple(new_es)

        e0, e1 = lax.fori_loop(0, myseq, t_body,
                               (jnp.int32(0), jnp.int32(0)))
        lin0 = _final_lin(_N_NUM // _L, 0)
        lin1 = _final_lin(_N_NUM // _L, _N_NUM)
        res = jnp.where(iota == 0, lin0,
              jnp.where(iota == 1, e0.astype(jnp.float32),
              jnp.where(iota == 2, lin1,
              jnp.where(iota == 3, e1.astype(jnp.float32), 0.0))))
        out_v[...] = res
        pltpu.sync_copy(out_v, out_ref.at[wid])


def kernel(est_llhs, seqlengths, den_src, den_dst, den_pdf, den_weight,
           den_start, den_final, num_src, num_dst, num_pdf, num_weight,
           num_start, num_final):
    llh_rows = est_llhs.reshape(_S * _B * _T, _C)
    seql16 = jnp.zeros((16,), jnp.int32).at[:_B].set(
        seqlengths.astype(jnp.int32))
    mesh = plsc.VectorSubcoreMesh(core_axis_name="c", subcore_axis_name="s",
                                  num_cores=_NC, num_subcores=_NS)
    scratch = [
        pltpu.VMEM((_A_DEN,), jnp.int32),
        pltpu.VMEM((_A_DEN,), jnp.int32),
        pltpu.VMEM((_A_DEN,), jnp.int32),
        pltpu.VMEM((_A_DEN,), jnp.int32),
        pltpu.VMEM((_A_DEN,), jnp.float32),
        pltpu.VMEM((_N_DEN,), jnp.float32),
        pltpu.VMEM((_N_DEN,), jnp.float32),
        pltpu.VMEM((_N_DEN,), jnp.float32),
        pltpu.VMEM((2 * _C,), jnp.float32),
        pltpu.VMEM((_C,), jnp.float32),
        pltpu.VMEM((16,), jnp.int32),
        pltpu.VMEM((16,), jnp.float32),
        pltpu.SemaphoreType.DMA,
        pltpu.SemaphoreType.DMA,
    ]
    run = pl.kernel(_sc_body,
                    out_type=jax.ShapeDtypeStruct((_NW, 16), jnp.float32),
                    mesh=mesh, scratch_types=scratch,
                    compiler_params=pltpu.CompilerParams(
                        needs_layout_passes=False))
    out = run(llh_rows, seql16,
              den_src.astype(jnp.int32), den_dst.astype(jnp.int32),
              den_pdf.astype(jnp.int32), den_weight,
              den_start, den_final,
              num_src.reshape(_S * _B, _A_NUM).astype(jnp.int32),
              num_dst.reshape(_S * _B, _A_NUM).astype(jnp.int32),
              num_pdf.reshape(_S * _B, _A_NUM).astype(jnp.int32),
              num_weight.reshape(_S * _B, _A_NUM),
              num_start.reshape(_S * _B, _N_NUM),
              num_final.reshape(_S * _B, _N_NUM))

    den = out[:_L]
    den_llh = (jnp.log(den[:, 0]) + den[:, 1] * _LN2).reshape(_S, _B).T
    num = out[_L:]
    num_ll = (jnp.log(num[:, jnp.array([0, 2])])
              + num[:, jnp.array([1, 3])] * _LN2).reshape(_S, _B, 2)
    nlp0 = jnp.stack([num_ll[0, :, 0], num_ll[1, :, 1]], axis=1)
    nlp1 = jnp.stack([num_ll[0, :, 1], num_ll[1, :, 0]], axis=1)
    use1 = jnp.sum(nlp1, axis=1) > jnp.sum(nlp0, axis=1)
    num_llh = jnp.where(use1[:, None], nlp1, nlp0)
    loss = -(num_llh - _DEN_SCALE * den_llh)
    return loss.sum()
